# baseline (speedup 1.0000x reference)
import jax
import jax.numpy as jnp
from jax import lax
from jax.experimental import pallas as pl
from jax.experimental.pallas import tpu as pltpu
from jax.experimental.pallas import tpu_sc as plsc

_INFO = plsc.get_sparse_core_info()
_NC = _INFO.num_cores
_NS = _INFO.num_subcores
_NW = _NC * _NS

_BATCH = 16384
_SEQ = 50
_B = _BATCH * _SEQ
_D = 64
_DP = 128
_BPW = _BATCH // _NW
_BLK = 128
_NQ = _BPW // _BLK
_NBLK = _SEQ * _NQ


def _transpose_block(gb, ob):
    for j0 in range(0, _BLK, 16):
        rows = j0 + lax.broadcasted_iota(jnp.int32, (16,), 0)
        for c in range(_D):
            cols = jnp.full((16,), c, jnp.int32)
            ob[c, pl.ds(j0, 16)] = plsc.load_gather(gb, [rows, cols])


def _body(tok_hbm, w_hbm, out_hbm, idxraw, idxs0, idxs1,
          gb0, gb1, ob0, ob1, gs0, gs1, os0, os1):
    idxs = (idxs0, idxs1)
    gbs = (gb0, gb1)
    obs = (ob0, ob1)
    gsems = (gs0, gs1)
    osems = (os0, os1)

    wid = lax.axis_index("s") * _NC + lax.axis_index("c")
    b0 = wid * _BPW
    pltpu.sync_copy(tok_hbm.at[pl.ds(b0 * _SEQ, _BPW * _SEQ)], idxraw)

    iota16 = lax.broadcasted_iota(jnp.int32, (16,), 0)
    iota_s = iota16 * _SEQ

    def build_and_fire(blk, p):
        s = lax.shift_right_logical(blk, 2)
        q = lax.bitwise_and(blk, 3)
        base = (q * _BLK) * _SEQ + s
        for j0 in range(0, _BLK, 16):
            pos = iota_s + (base + j0 * _SEQ)
            idxs[p][pl.ds(j0, 16)] = plsc.load_gather(idxraw, [pos])
        pltpu.async_copy(w_hbm.at[idxs[p]], gbs[p], gsems[p])

    build_and_fire(0, 0)
    build_and_fire(1, 1)

    def outer(t, carry):
        for p in range(2):
            blk = 2 * t + p
            pltpu.make_async_copy(
                w_hbm.at[idxs[p]], gbs[p], gsems[p]).wait()

            @pl.when(blk >= 2)
            def _():
                pltpu.make_async_copy(
                    obs[p], out_hbm.at[0, :, pl.ds(0, _BLK)],
                    osems[p]).wait()

            _transpose_block(gbs[p], obs[p])

            s = lax.shift_right_logical(blk, 2)
            q = lax.bitwise_and(blk, 3)
            pltpu.async_copy(
                obs[p],
                out_hbm.at[s, :, pl.ds(b0 + q * _BLK, _BLK)],
                osems[p])

            @pl.when(blk + 2 < _NBLK)
            def _():
                build_and_fire(blk + 2, p)
        return carry

    lax.fori_loop(0, _NBLK // 2, outer, 0)

    for p in range(2):
        pltpu.make_async_copy(
            obs[p], out_hbm.at[0, :, pl.ds(0, _BLK)], osems[p]).wait()


def kernel(token_ids, weight):
    tok = token_ids.reshape(_B)
    wpad = jnp.pad(weight, ((0, 0), (0, _DP - _D)))
    mesh = plsc.VectorSubcoreMesh(core_axis_name="c", subcore_axis_name="s")
    out_t = pl.kernel(
        _body,
        mesh=mesh,
        compiler_params=pltpu.CompilerParams(
            use_tc_tiling_on_sc=True, needs_layout_passes=False),
        out_type=jax.ShapeDtypeStruct((_SEQ, _D, _BATCH), jnp.float32),
        scratch_types=[
            pltpu.VMEM((_BPW * _SEQ,), jnp.int32),
            pltpu.VMEM((_BLK,), jnp.int32),
            pltpu.VMEM((_BLK,), jnp.int32),
            pltpu.VMEM((_BLK, _DP), jnp.float32),
            pltpu.VMEM((_BLK, _DP), jnp.float32),
            pltpu.VMEM((_D, _BLK), jnp.float32),
            pltpu.VMEM((_D, _BLK), jnp.float32),
        ] + [pltpu.SemaphoreType.DMA for _ in range(4)],
    )(tok, wpad)
    return jnp.transpose(out_t, (2, 0, 1))


# speedup vs baseline: 1.5303x; 1.5303x over previous
import jax
import jax.numpy as jnp
from jax import lax
from jax.experimental import pallas as pl
from jax.experimental.pallas import tpu as pltpu
from jax.experimental.pallas import tpu_sc as plsc

_INFO = plsc.get_sparse_core_info()
_NC = _INFO.num_cores
_NS = _INFO.num_subcores
_NW = _NC * _NS

_BATCH = 16384
_SEQ = 50
_B = _BATCH * _SEQ
_D = 64
_DP = 128
_BPW = _BATCH // _NW
_BLK = 128
_NQ = _BPW // _BLK
_NBLK = _SEQ * _NQ


def _transpose_block(gb, ob, row_vecs):
    @plsc.parallel_loop(0, _D, unroll=8)
    def _(c):
        cols = jnp.full((16,), c, jnp.int32)
        for k in range(_BLK // 16):
            ob[c, pl.ds(k * 16, 16)] = plsc.load_gather(
                gb, [row_vecs[k], cols])


def _body(tok_hbm, w_hbm, out_hbm, idxraw, idxs0, idxs1,
          gb0, gb1, ob0, ob1, gs0, gs1, os0, os1):
    idxs = (idxs0, idxs1)
    gbs = (gb0, gb1)
    obs = (ob0, ob1)
    gsems = (gs0, gs1)
    osems = (os0, os1)

    wid = lax.axis_index("s") * _NC + lax.axis_index("c")
    b0 = wid * _BPW
    pltpu.sync_copy(tok_hbm.at[pl.ds(b0 * _SEQ, _BPW * _SEQ)], idxraw)

    iota16 = lax.broadcasted_iota(jnp.int32, (16,), 0)
    iota_s = iota16 * _SEQ
    row_vecs = [k * 16 + iota16 for k in range(_BLK // 16)]

    def build_and_fire(blk, p):
        s = lax.shift_right_logical(blk, 2)
        q = lax.bitwise_and(blk, 3)
        base = (q * _BLK) * _SEQ + s
        for j0 in range(0, _BLK, 16):
            pos = iota_s + (base + j0 * _SEQ)
            idxs[p][pl.ds(j0, 16)] = plsc.load_gather(idxraw, [pos])
        pltpu.async_copy(w_hbm.at[idxs[p]], gbs[p], gsems[p])

    build_and_fire(0, 0)
    build_and_fire(1, 1)

    def outer(t, carry):
        for p in range(2):
            blk = 2 * t + p
            pltpu.make_async_copy(
                w_hbm.at[idxs[p]], gbs[p], gsems[p]).wait()

            @pl.when(blk >= 2)
            def _():
                pltpu.make_async_copy(
                    obs[p], out_hbm.at[0, :, pl.ds(0, _BLK)],
                    osems[p]).wait()

            _transpose_block(gbs[p], obs[p], row_vecs)

            s = lax.shift_right_logical(blk, 2)
            q = lax.bitwise_and(blk, 3)
            pltpu.async_copy(
                obs[p],
                out_hbm.at[s, :, pl.ds(b0 + q * _BLK, _BLK)],
                osems[p])

            @pl.when(blk + 2 < _NBLK)
            def _():
                build_and_fire(blk + 2, p)
        return carry

    lax.fori_loop(0, _NBLK // 2, outer, 0)

    for p in range(2):
        pltpu.make_async_copy(
            obs[p], out_hbm.at[0, :, pl.ds(0, _BLK)], osems[p]).wait()


def kernel(token_ids, weight):
    tok = token_ids.reshape(_B)
    wpad = jnp.pad(weight, ((0, 0), (0, _DP - _D)))
    mesh = plsc.VectorSubcoreMesh(core_axis_name="c", subcore_axis_name="s")
    out_t = pl.kernel(
        _body,
        mesh=mesh,
        compiler_params=pltpu.CompilerParams(
            use_tc_tiling_on_sc=True, needs_layout_passes=False),
        out_type=jax.ShapeDtypeStruct((_SEQ, _D, _BATCH), jnp.float32),
        scratch_types=[
            pltpu.VMEM((_BPW * _SEQ,), jnp.int32),
            pltpu.VMEM((_BLK,), jnp.int32),
            pltpu.VMEM((_BLK,), jnp.int32),
            pltpu.VMEM((_BLK, _DP), jnp.float32),
            pltpu.VMEM((_BLK, _DP), jnp.float32),
            pltpu.VMEM((_D, _BLK), jnp.float32),
            pltpu.VMEM((_D, _BLK), jnp.float32),
        ] + [pltpu.SemaphoreType.DMA for _ in range(4)],
    )(tok, wpad)
    return jnp.transpose(out_t, (2, 0, 1))
